# baseline (speedup 1.0000x reference)
import functools

import jax
import jax.numpy as jnp
from jax import lax
from jax.experimental import pallas as pl
from jax.experimental.pallas import tpu as pltpu
from jax.experimental.pallas import tpu_sc as plsc

_L = 16
_G = 128
_C = 512


@functools.lru_cache(maxsize=None)
def _build_gather(batch: int, hist: int, vocab2: int, dim: int):
    info = plsc.get_sparse_core_info()
    nc, ns = info.num_cores, info.num_subcores
    nw = nc * ns
    assert batch % _C == 0 and dim % _L == 0
    cpt = batch // _C
    total = hist * cpt
    assert total % (2 * nw) == 0
    npair = total // (2 * nw)
    ng = _C // _G

    mesh = plsc.VectorSubcoreMesh(core_axis_name="c", subcore_axis_name="s")

    @functools.partial(
        pl.kernel,
        out_type=jax.ShapeDtypeStruct((hist, dim, batch), jnp.float32),
        mesh=mesh,
        compiler_params=pltpu.CompilerParams(
            use_tc_tiling_on_sc=False, needs_layout_passes=False),
        scratch_types=[
            pltpu.VMEM((_C,), jnp.int32),
            pltpu.VMEM((_C, dim), jnp.float32),
            pltpu.VMEM((_C,), jnp.int32),
            pltpu.VMEM((_C, dim), jnp.float32),
            pltpu.VMEM((dim, _C), jnp.float32),
            pltpu.VMEM((_L,), jnp.float32),
            pltpu.SemaphoreType.DMA,
            pltpu.SemaphoreType.DMA,
        ],
    )
    def body(idx_hbm, table_hbm, zs_hbm, out_hbm,
             idx_a, rows_a, idx_b, rows_b, trans_v, zs_v, sem_a, sem_b):
        wid = lax.axis_index("s") * nc + lax.axis_index("c")
        c0 = wid * (2 * npair)
        pltpu.sync_copy(zs_hbm, zs_v)
        iotav = jnp.arange(_L, dtype=jnp.int32)
        rowvecs = [iotav + (cg * _L) for cg in range(dim // _L)]

        def issue(idx_v, rows_v, sem, cid):
            base = pl.multiple_of(cid * _C, _C)
            pltpu.sync_copy(idx_hbm.at[pl.ds(base, _C)], idx_v)
            for k in range(ng):
                pltpu.async_copy(
                    table_hbm.at[idx_v.at[pl.ds(k * _G, _G)]],
                    rows_v.at[pl.ds(k * _G, _G)],
                    sem,
                )

        def finish(idx_v, rows_v, sem, cid):
            for k in range(ng):
                pltpu.make_async_copy(
                    table_hbm.at[idx_v.at[pl.ds(k * _G, _G)]],
                    rows_v.at[pl.ds(k * _G, _G)],
                    sem,
                ).wait()

            @pl.when(cid % cpt == 0)
            def _fix():
                zs = zs_v[...]
                for k in range(dim // _L):
                    sl = pl.ds(k * _L, _L)
                    rows_v[0, sl] = rows_v[0, sl] * zs

            def ti(i, carry):
                colv = jnp.full((_L,), i, dtype=jnp.int32)
                for cg in range(dim // _L):
                    x = rows_v[i, pl.ds(cg * _L, _L)]
                    plsc.store_scatter(trans_v, [rowvecs[cg], colv], x)
                return carry

            lax.fori_loop(0, _C, ti, 0)

            t = cid // cpt
            i0 = pl.multiple_of((cid % cpt) * _C, _C)
            pltpu.sync_copy(trans_v, out_hbm.at[t, :, pl.ds(i0, _C)])

        issue(idx_a, rows_a, sem_a, c0)

        def pair(j, carry):
            e = c0 + 2 * j
            issue(idx_b, rows_b, sem_b, e + 1)
            finish(idx_a, rows_a, sem_a, e)

            @pl.when(j < npair - 1)
            def _next():
                issue(idx_a, rows_a, sem_a, e + 2)

            finish(idx_b, rows_b, sem_b, e + 1)
            return carry

        lax.fori_loop(0, npair, pair, 0)

    return body


def kernel(y, table, is_train):
    b, h = y.shape
    vocab, dim = table.shape
    yt = y.T.astype(jnp.int32)
    shifted = jnp.concatenate([jnp.zeros((h, 1), jnp.int32), yt[:, :-1]], axis=1)
    train = is_train != 0
    idx = jnp.where(train, shifted, yt).reshape(-1) * 2
    zscale = jnp.where(train, jnp.zeros((_L,), jnp.float32),
                       jnp.ones((_L,), jnp.float32))
    table2 = jnp.pad(table, ((0, 0), (0, dim))).reshape(2 * vocab, dim)
    out_t = _build_gather(b, h, 2 * vocab, dim)(idx, table2, zscale)
    return jnp.transpose(out_t, (2, 0, 1))


# speedup vs baseline: 2.0680x; 2.0680x over previous
import functools

import jax
import jax.numpy as jnp
from jax import lax
from jax.experimental import pallas as pl
from jax.experimental.pallas import tpu as pltpu
from jax.experimental.pallas import tpu_sc as plsc

_L = 16
_G = 128
_C = 512
_P = 128


@functools.lru_cache(maxsize=None)
def _build_gather(n_rows: int, vocab2: int, dim: int, hist: int):
    info = plsc.get_sparse_core_info()
    nc, ns = info.num_cores, info.num_subcores
    nw = nc * ns
    assert n_rows % (2 * nw * _C) == 0 and dim % _L == 0
    rpw = n_rows // nw
    npair = rpw // (2 * _C)
    ng = _C // _G

    mesh = plsc.VectorSubcoreMesh(core_axis_name="c", subcore_axis_name="s")

    @functools.partial(
        pl.kernel,
        out_type=jax.ShapeDtypeStruct((n_rows, _P), jnp.float32),
        mesh=mesh,
        compiler_params=pltpu.CompilerParams(
            use_tc_tiling_on_sc=False, needs_layout_passes=False),
        scratch_types=[
            pltpu.VMEM((_C,), jnp.int32),
            pltpu.VMEM((_C, dim), jnp.float32),
            pltpu.VMEM((_C,), jnp.int32),
            pltpu.VMEM((_C, dim), jnp.float32),
            pltpu.VMEM((_L,), jnp.float32),
            pltpu.SemaphoreType.DMA,
            pltpu.SemaphoreType.DMA,
        ],
    )
    def body(idx_hbm, table_hbm, zs_hbm, out_hbm,
             idx_a, rows_a, idx_b, rows_b, zs_v, sem_a, sem_b):
        wid = lax.axis_index("s") * nc + lax.axis_index("c")
        w0 = wid * rpw
        pltpu.sync_copy(zs_hbm, zs_v)

        def issue(idx_v, rows_v, sem, base):
            pltpu.sync_copy(idx_hbm.at[pl.ds(base, _C)], idx_v)
            for k in range(ng):
                pltpu.async_copy(
                    table_hbm.at[idx_v.at[pl.ds(k * _G, _G)]],
                    rows_v.at[pl.ds(k * _G, _G)],
                    sem,
                )

        def finish(idx_v, rows_v, sem, base, first):
            for k in range(ng):
                pltpu.make_async_copy(
                    table_hbm.at[idx_v.at[pl.ds(k * _G, _G)]],
                    rows_v.at[pl.ds(k * _G, _G)],
                    sem,
                ).wait()

            @pl.when(first)
            def _fix():
                zs = zs_v[...]

                def rowfix(i, c2):
                    for k in range(dim // _L):
                        sl = pl.ds(k * _L, _L)
                        rows_v[i, sl] = rows_v[i, sl] * zs
                    return c2

                lax.fori_loop(0, hist, rowfix, 0)

            pltpu.sync_copy(rows_v,
                            out_hbm.at[pl.ds(base, _C), pl.ds(0, dim)])

        issue(idx_a, rows_a, sem_a, pl.multiple_of(w0, _C))

        def pair(j, carry):
            e_base = pl.multiple_of(w0 + (2 * j) * _C, _C)
            o_base = pl.multiple_of(w0 + (2 * j + 1) * _C, _C)
            issue(idx_b, rows_b, sem_b, o_base)
            finish(idx_a, rows_a, sem_a, e_base, (wid == 0) & (j == 0))

            @pl.when(j < npair - 1)
            def _next():
                issue(idx_a, rows_a, sem_a,
                      pl.multiple_of(w0 + (2 * j + 2) * _C, _C))

            finish(idx_b, rows_b, sem_b, o_base, False)
            return carry

        lax.fori_loop(0, npair, pair, 0)

    return body


def kernel(y, table, is_train):
    b, h = y.shape
    vocab, dim = table.shape
    flat = y.reshape(-1).astype(jnp.int32)
    shifted = jnp.concatenate([jnp.zeros((h,), jnp.int32), flat[:-h]])
    train = is_train != 0
    idx = jnp.where(train, shifted, flat) * 2
    zscale = jnp.where(train, jnp.zeros((_L,), jnp.float32),
                       jnp.ones((_L,), jnp.float32))
    table2 = jnp.pad(table, ((0, 0), (0, _P - dim))).reshape(2 * vocab, dim)
    out128 = _build_gather(b * h, 2 * vocab, dim, h)(idx, table2, zscale)
    return out128.reshape(b, h, _P)[:, :, :dim]


# speedup vs baseline: 2.0705x; 1.0012x over previous
import functools

import jax
import jax.numpy as jnp
from jax import lax
from jax.experimental import pallas as pl
from jax.experimental.pallas import tpu as pltpu
from jax.experimental.pallas import tpu_sc as plsc

_L = 16
_G = 256
_C = 512
_P = 128


@functools.lru_cache(maxsize=None)
def _build_gather(n_rows: int, vocab2: int, dim: int, hist: int):
    info = plsc.get_sparse_core_info()
    nc, ns = info.num_cores, info.num_subcores
    nw = nc * ns
    assert n_rows % (2 * nw * _C) == 0 and dim % _L == 0
    rpw = n_rows // nw
    npair = rpw // (2 * _C)
    ng = _C // _G

    mesh = plsc.VectorSubcoreMesh(core_axis_name="c", subcore_axis_name="s")

    @functools.partial(
        pl.kernel,
        out_type=jax.ShapeDtypeStruct((n_rows, _P), jnp.float32),
        mesh=mesh,
        compiler_params=pltpu.CompilerParams(
            use_tc_tiling_on_sc=False, needs_layout_passes=False),
        scratch_types=[
            pltpu.VMEM((_C,), jnp.int32),
            pltpu.VMEM((_C, dim), jnp.float32),
            pltpu.VMEM((_C,), jnp.int32),
            pltpu.VMEM((_C, dim), jnp.float32),
            pltpu.VMEM((_L,), jnp.float32),
            pltpu.SemaphoreType.DMA,
            pltpu.SemaphoreType.DMA,
        ],
    )
    def body(idx_hbm, table_hbm, zs_hbm, out_hbm,
             idx_a, rows_a, idx_b, rows_b, zs_v, sem_a, sem_b):
        wid = lax.axis_index("s") * nc + lax.axis_index("c")
        w0 = wid * rpw
        pltpu.sync_copy(zs_hbm, zs_v)

        def issue(idx_v, rows_v, sem, base):
            pltpu.sync_copy(idx_hbm.at[pl.ds(base, _C)], idx_v)
            for k in range(ng):
                pltpu.async_copy(
                    table_hbm.at[idx_v.at[pl.ds(k * _G, _G)]],
                    rows_v.at[pl.ds(k * _G, _G)],
                    sem,
                )

        def finish(idx_v, rows_v, sem, base, first):
            for k in range(ng):
                pltpu.make_async_copy(
                    table_hbm.at[idx_v.at[pl.ds(k * _G, _G)]],
                    rows_v.at[pl.ds(k * _G, _G)],
                    sem,
                ).wait()

            @pl.when(first)
            def _fix():
                zs = zs_v[...]

                def rowfix(i, c2):
                    for k in range(dim // _L):
                        sl = pl.ds(k * _L, _L)
                        rows_v[i, sl] = rows_v[i, sl] * zs
                    return c2

                lax.fori_loop(0, hist, rowfix, 0)

            pltpu.sync_copy(rows_v,
                            out_hbm.at[pl.ds(base, _C), pl.ds(0, dim)])

        issue(idx_a, rows_a, sem_a, pl.multiple_of(w0, _C))

        def pair(j, carry):
            e_base = pl.multiple_of(w0 + (2 * j) * _C, _C)
            o_base = pl.multiple_of(w0 + (2 * j + 1) * _C, _C)
            issue(idx_b, rows_b, sem_b, o_base)
            finish(idx_a, rows_a, sem_a, e_base, (wid == 0) & (j == 0))

            @pl.when(j < npair - 1)
            def _next():
                issue(idx_a, rows_a, sem_a,
                      pl.multiple_of(w0 + (2 * j + 2) * _C, _C))

            finish(idx_b, rows_b, sem_b, o_base, False)
            return carry

        lax.fori_loop(0, npair, pair, 0)

    return body


def kernel(y, table, is_train):
    b, h = y.shape
    vocab, dim = table.shape
    flat = y.reshape(-1).astype(jnp.int32)
    shifted = jnp.concatenate([jnp.zeros((h,), jnp.int32), flat[:-h]])
    train = is_train != 0
    idx = jnp.where(train, shifted, flat) * 2
    zscale = jnp.where(train, jnp.zeros((_L,), jnp.float32),
                       jnp.ones((_L,), jnp.float32))
    table2 = jnp.pad(table, ((0, 0), (0, _P - dim))).reshape(2 * vocab, dim)
    out128 = _build_gather(b * h, 2 * vocab, dim, h)(idx, table2, zscale)
    return out128.reshape(b, h, _P)[:, :, :dim]
